# baseline (speedup 1.0000x reference)
import jax
import jax.numpy as jnp
from jax import lax
from jax.experimental import pallas as pl
from jax.experimental.pallas import tpu as pltpu
from jax.experimental.pallas import tpu_sc as plsc

N = 160000
D = 256
NUM_REL = 64
NC = 2
NS = 16
NW = NC * NS
ROWS_PER_W = N // NW
CHUNK = 40
NCHUNK = ROWS_PER_W // CHUNK
NBUF = 2
LANES = 16
DSTEPS = D // LANES


def _body(src_hbm, dst_hbm, ridx_hbm, emb_hbm, scale_hbm, out_hbm,
          idx_v, scal_v, rvec, src_buf, dst_buf, out_v, tile_v, sems, rsem):
    wid = lax.axis_index("s") * NC + lax.axis_index("c")
    base = wid * ROWS_PER_W

    pltpu.sync_copy(ridx_hbm, idx_v)
    pltpu.sync_copy(scale_hbm, scal_v)
    pltpu.async_copy(emb_hbm.at[idx_v], rvec, rsem).wait()
    s_vec = scal_v[pl.ds(0, LANES)]
    for j in range(DSTEPS):
        rvec[0, pl.ds(j * LANES, LANES)] = rvec[0, pl.ds(j * LANES, LANES)] * s_vec
    r_regs = [rvec[0, pl.ds(j * LANES, LANES)] for j in range(DSTEPS)]

    def start(c, b):
        rows = pl.ds(base + c * CHUNK, CHUNK)
        pltpu.make_async_copy(src_hbm.at[rows], src_buf.at[b], sems.at[b]).start()
        pltpu.make_async_copy(dst_hbm.at[rows], dst_buf.at[b], sems.at[b]).start()

    def wait(c, b):
        rows = pl.ds(base + c * CHUNK, CHUNK)
        pltpu.make_async_copy(src_hbm.at[rows], src_buf.at[b], sems.at[b]).wait()
        pltpu.make_async_copy(dst_hbm.at[rows], dst_buf.at[b], sems.at[b]).wait()

    for b in range(NBUF):
        start(b, b)

    lane_ids = lax.iota(jnp.int32, LANES)

    def row_acc(sbuf, dbuf, i):
        acc = sbuf[i, pl.ds(0, LANES)] * r_regs[0] * dbuf[i, pl.ds(0, LANES)]
        for j in range(1, DSTEPS):
            sl = pl.ds(j * LANES, LANES)
            acc = acc + sbuf[i, sl] * r_regs[j] * dbuf[i, sl]
        return acc

    def tile_row_sums(tile):
        sv = None
        for j in range(DSTEPS):
            col = plsc.load_gather(
                tile, [lane_ids, jnp.full((LANES,), j, jnp.int32)])
            sv = col if sv is None else sv + col
        return sv

    NFULL = CHUNK // LANES
    NREM = CHUNK % LANES

    def compute_chunk(c, b):
        sbuf = src_buf.at[b]
        dbuf = dst_buf.at[b]
        def group_body(g, _):
            ro = g * LANES
            for k in range(LANES):
                tile_v[k, pl.ds(0, LANES)] = row_acc(sbuf, dbuf, ro + k)
            sv = tile_row_sums(tile_v)
            plsc.store_scatter(out_v, [lane_ids + c * CHUNK + ro], sv)
            return 0

        lax.fori_loop(0, NFULL, group_body, 0)
        if NREM:
            ro = NFULL * LANES
            for k in range(NREM):
                tile_v[k, pl.ds(0, LANES)] = row_acc(sbuf, dbuf, ro + k)
            sv = tile_row_sums(tile_v)
            plsc.store_scatter(out_v, [lane_ids + (c * CHUNK + ro)], sv,
                               mask=lane_ids < NREM)

    def chunk_pair(it, carry):
        c0 = it * NBUF
        for b in range(NBUF):
            c = c0 + b
            wait(c, b)
            compute_chunk(c, b)

            @pl.when(c + NBUF < NCHUNK)
            def _(c=c, b=b):
                start(c + NBUF, b)
        return carry

    lax.fori_loop(0, NCHUNK // NBUF, chunk_pair, 0)
    for c in range(NCHUNK - NCHUNK % NBUF, NCHUNK):
        b = c % NBUF
        wait(c, b)
        compute_chunk(c, b)
    pltpu.sync_copy(out_v, out_hbm.at[pl.ds(base, ROWS_PER_W)])


BN = 256


def _tc_body(idx_ref, scale_ref, emb_ref, src_ref, dst_ref, out_ref):
    r = emb_ref[idx_ref[0], :] * scale_ref[0]
    out_ref[:] = jnp.sum(src_ref[:] * r[None, :] * dst_ref[:], axis=1)


def _tc_kernel(z_src, z_dst, ridx, rel_emb, scale, n_rows, row0):
    grid = (n_rows // BN,)
    return pl.pallas_call(
        _tc_body,
        grid=grid,
        in_specs=[
            pl.BlockSpec(memory_space=pltpu.SMEM),
            pl.BlockSpec(memory_space=pltpu.SMEM),
            pl.BlockSpec((NUM_REL, D), lambda i: (0, 0)),
            pl.BlockSpec((BN, D), lambda i, row0=row0: (row0 // BN + i, 0)),
            pl.BlockSpec((BN, D), lambda i, row0=row0: (row0 // BN + i, 0)),
        ],
        out_specs=pl.BlockSpec((BN,), lambda i: (i,)),
        out_shape=jax.ShapeDtypeStruct((n_rows,), jnp.float32),
    )(ridx, scale, rel_emb, z_src, z_dst)


def kernel(z_src, z_dst, rel_idx, rel_emb, score_scale):
    ridx = jnp.asarray(rel_idx, jnp.int32).reshape((1,))
    scale1 = jnp.asarray(score_scale, jnp.float32).reshape((1,))
    return _tc_kernel(z_src, z_dst, ridx, rel_emb, scale1, N, 0)


def _unused_sc_kernel(z_src, z_dst, rel_idx, rel_emb, score_scale):
    ridx = jnp.asarray(rel_idx, jnp.int32).reshape((1,))
    scale = jnp.full((LANES,), score_scale, jnp.float32)
    mesh = plsc.VectorSubcoreMesh(
        core_axis_name="c", subcore_axis_name="s",
        num_cores=NC, num_subcores=NS,
    )
    f = pl.kernel(
        _body,
        out_type=jax.ShapeDtypeStruct((N,), jnp.float32),
        mesh=mesh,
        scratch_types=[
            pltpu.VMEM((1,), jnp.int32),
            pltpu.VMEM((LANES,), jnp.float32),
            pltpu.VMEM((1, D), jnp.float32),
            pltpu.VMEM((NBUF, CHUNK, D), jnp.float32),
            pltpu.VMEM((NBUF, CHUNK, D), jnp.float32),
            pltpu.VMEM((ROWS_PER_W,), jnp.float32),
            pltpu.VMEM((LANES, LANES), jnp.float32),
            pltpu.SemaphoreType.DMA((NBUF,)),
            pltpu.SemaphoreType.DMA,
        ],
        compiler_params=pltpu.CompilerParams(needs_layout_passes=False),
    )
    return f(z_src, z_dst, ridx, rel_emb, scale)


# speedup vs baseline: 1.0013x; 1.0013x over previous
import jax
import jax.numpy as jnp
from jax import lax
from jax.experimental import pallas as pl
from jax.experimental.pallas import tpu as pltpu
from jax.experimental.pallas import tpu_sc as plsc

N = 160000
D = 256
NUM_REL = 64
NC = 2
NS = 16
NW = NC * NS
ROWS_PER_W = N // NW
CHUNK = 40
NCHUNK = ROWS_PER_W // CHUNK
NBUF = 2
LANES = 16
DSTEPS = D // LANES


def _body(src_hbm, dst_hbm, ridx_hbm, emb_hbm, scale_hbm, out_hbm,
          idx_v, scal_v, rvec, src_buf, dst_buf, out_v, tile_v, sems, rsem):
    wid = lax.axis_index("s") * NC + lax.axis_index("c")
    base = wid * ROWS_PER_W

    pltpu.sync_copy(ridx_hbm, idx_v)
    pltpu.sync_copy(scale_hbm, scal_v)
    pltpu.async_copy(emb_hbm.at[idx_v], rvec, rsem).wait()
    s_vec = scal_v[pl.ds(0, LANES)]
    for j in range(DSTEPS):
        rvec[0, pl.ds(j * LANES, LANES)] = rvec[0, pl.ds(j * LANES, LANES)] * s_vec
    r_regs = [rvec[0, pl.ds(j * LANES, LANES)] for j in range(DSTEPS)]

    def start(c, b):
        rows = pl.ds(base + c * CHUNK, CHUNK)
        pltpu.make_async_copy(src_hbm.at[rows], src_buf.at[b], sems.at[b]).start()
        pltpu.make_async_copy(dst_hbm.at[rows], dst_buf.at[b], sems.at[b]).start()

    def wait(c, b):
        rows = pl.ds(base + c * CHUNK, CHUNK)
        pltpu.make_async_copy(src_hbm.at[rows], src_buf.at[b], sems.at[b]).wait()
        pltpu.make_async_copy(dst_hbm.at[rows], dst_buf.at[b], sems.at[b]).wait()

    for b in range(NBUF):
        start(b, b)

    lane_ids = lax.iota(jnp.int32, LANES)

    def row_acc(sbuf, dbuf, i):
        acc = sbuf[i, pl.ds(0, LANES)] * r_regs[0] * dbuf[i, pl.ds(0, LANES)]
        for j in range(1, DSTEPS):
            sl = pl.ds(j * LANES, LANES)
            acc = acc + sbuf[i, sl] * r_regs[j] * dbuf[i, sl]
        return acc

    def tile_row_sums(tile):
        sv = None
        for j in range(DSTEPS):
            col = plsc.load_gather(
                tile, [lane_ids, jnp.full((LANES,), j, jnp.int32)])
            sv = col if sv is None else sv + col
        return sv

    NFULL = CHUNK // LANES
    NREM = CHUNK % LANES

    def compute_chunk(c, b):
        sbuf = src_buf.at[b]
        dbuf = dst_buf.at[b]
        def group_body(g, _):
            ro = g * LANES
            for k in range(LANES):
                tile_v[k, pl.ds(0, LANES)] = row_acc(sbuf, dbuf, ro + k)
            sv = tile_row_sums(tile_v)
            plsc.store_scatter(out_v, [lane_ids + c * CHUNK + ro], sv)
            return 0

        lax.fori_loop(0, NFULL, group_body, 0)
        if NREM:
            ro = NFULL * LANES
            for k in range(NREM):
                tile_v[k, pl.ds(0, LANES)] = row_acc(sbuf, dbuf, ro + k)
            sv = tile_row_sums(tile_v)
            plsc.store_scatter(out_v, [lane_ids + (c * CHUNK + ro)], sv,
                               mask=lane_ids < NREM)

    def chunk_pair(it, carry):
        c0 = it * NBUF
        for b in range(NBUF):
            c = c0 + b
            wait(c, b)
            compute_chunk(c, b)

            @pl.when(c + NBUF < NCHUNK)
            def _(c=c, b=b):
                start(c + NBUF, b)
        return carry

    lax.fori_loop(0, NCHUNK // NBUF, chunk_pair, 0)
    for c in range(NCHUNK - NCHUNK % NBUF, NCHUNK):
        b = c % NBUF
        wait(c, b)
        compute_chunk(c, b)
    pltpu.sync_copy(out_v, out_hbm.at[pl.ds(base, ROWS_PER_W)])


BN = 256


def _tc_body(idx_ref, scale_ref, emb_ref, src_ref, dst_ref, out_ref):
    r = emb_ref[idx_ref[0], :] * scale_ref[0]
    t = src_ref[:] * dst_ref[:]
    out_ref[:] = jax.lax.dot_general(
        t, r, (((1,), (0,)), ((), ())),
        preferred_element_type=jnp.float32)


def _tc_kernel(z_src, z_dst, ridx, rel_emb, scale, n_rows, row0):
    grid = (n_rows // BN,)
    return pl.pallas_call(
        _tc_body,
        grid=grid,
        in_specs=[
            pl.BlockSpec(memory_space=pltpu.SMEM),
            pl.BlockSpec(memory_space=pltpu.SMEM),
            pl.BlockSpec((NUM_REL, D), lambda i: (0, 0)),
            pl.BlockSpec((BN, D), lambda i, row0=row0: (row0 // BN + i, 0)),
            pl.BlockSpec((BN, D), lambda i, row0=row0: (row0 // BN + i, 0)),
        ],
        out_specs=pl.BlockSpec((BN,), lambda i: (i,)),
        out_shape=jax.ShapeDtypeStruct((n_rows,), jnp.float32),
    )(ridx, scale, rel_emb, z_src, z_dst)


def kernel(z_src, z_dst, rel_idx, rel_emb, score_scale):
    ridx = jnp.asarray(rel_idx, jnp.int32).reshape((1,))
    scale1 = jnp.asarray(score_scale, jnp.float32).reshape((1,))
    return _tc_kernel(z_src, z_dst, ridx, rel_emb, scale1, N, 0)


def _unused_sc_kernel(z_src, z_dst, rel_idx, rel_emb, score_scale):
    ridx = jnp.asarray(rel_idx, jnp.int32).reshape((1,))
    scale = jnp.full((LANES,), score_scale, jnp.float32)
    mesh = plsc.VectorSubcoreMesh(
        core_axis_name="c", subcore_axis_name="s",
        num_cores=NC, num_subcores=NS,
    )
    f = pl.kernel(
        _body,
        out_type=jax.ShapeDtypeStruct((N,), jnp.float32),
        mesh=mesh,
        scratch_types=[
            pltpu.VMEM((1,), jnp.int32),
            pltpu.VMEM((LANES,), jnp.float32),
            pltpu.VMEM((1, D), jnp.float32),
            pltpu.VMEM((NBUF, CHUNK, D), jnp.float32),
            pltpu.VMEM((NBUF, CHUNK, D), jnp.float32),
            pltpu.VMEM((ROWS_PER_W,), jnp.float32),
            pltpu.VMEM((LANES, LANES), jnp.float32),
            pltpu.SemaphoreType.DMA((NBUF,)),
            pltpu.SemaphoreType.DMA,
        ],
        compiler_params=pltpu.CompilerParams(needs_layout_passes=False),
    )
    return f(z_src, z_dst, ridx, rel_emb, scale)


# speedup vs baseline: 4.3138x; 4.3081x over previous
import jax
import jax.numpy as jnp
from jax import lax
from jax.experimental import pallas as pl
from jax.experimental.pallas import tpu as pltpu
from jax.experimental.pallas import tpu_sc as plsc

N = 160000
D = 256
NUM_REL = 64
NC = 2
NS = 16
NW = NC * NS
ROWS_PER_W = N // NW
CHUNK = 40
NCHUNK = ROWS_PER_W // CHUNK
NBUF = 2
LANES = 16
DSTEPS = D // LANES


def _body(src_hbm, dst_hbm, ridx_hbm, emb_hbm, scale_hbm, out_hbm,
          idx_v, scal_v, rvec, src_buf, dst_buf, out_v, tile_v, sems, rsem):
    wid = lax.axis_index("s") * NC + lax.axis_index("c")
    base = wid * ROWS_PER_W

    pltpu.sync_copy(ridx_hbm, idx_v)
    pltpu.sync_copy(scale_hbm, scal_v)
    pltpu.async_copy(emb_hbm.at[idx_v], rvec, rsem).wait()
    s_vec = scal_v[pl.ds(0, LANES)]
    for j in range(DSTEPS):
        rvec[0, pl.ds(j * LANES, LANES)] = rvec[0, pl.ds(j * LANES, LANES)] * s_vec
    r_regs = [rvec[0, pl.ds(j * LANES, LANES)] for j in range(DSTEPS)]

    def start(c, b):
        rows = pl.ds(base + c * CHUNK, CHUNK)
        pltpu.make_async_copy(src_hbm.at[rows], src_buf.at[b], sems.at[b]).start()
        pltpu.make_async_copy(dst_hbm.at[rows], dst_buf.at[b], sems.at[b]).start()

    def wait(c, b):
        rows = pl.ds(base + c * CHUNK, CHUNK)
        pltpu.make_async_copy(src_hbm.at[rows], src_buf.at[b], sems.at[b]).wait()
        pltpu.make_async_copy(dst_hbm.at[rows], dst_buf.at[b], sems.at[b]).wait()

    for b in range(NBUF):
        start(b, b)

    lane_ids = lax.iota(jnp.int32, LANES)

    def row_acc(sbuf, dbuf, i):
        acc = sbuf[i, pl.ds(0, LANES)] * r_regs[0] * dbuf[i, pl.ds(0, LANES)]
        for j in range(1, DSTEPS):
            sl = pl.ds(j * LANES, LANES)
            acc = acc + sbuf[i, sl] * r_regs[j] * dbuf[i, sl]
        return acc

    def tile_row_sums(tile):
        sv = None
        for j in range(DSTEPS):
            col = plsc.load_gather(
                tile, [lane_ids, jnp.full((LANES,), j, jnp.int32)])
            sv = col if sv is None else sv + col
        return sv

    NFULL = CHUNK // LANES
    NREM = CHUNK % LANES

    def compute_chunk(c, b):
        sbuf = src_buf.at[b]
        dbuf = dst_buf.at[b]
        def group_body(g, _):
            ro = g * LANES
            for k in range(LANES):
                tile_v[k, pl.ds(0, LANES)] = row_acc(sbuf, dbuf, ro + k)
            sv = tile_row_sums(tile_v)
            plsc.store_scatter(out_v, [lane_ids + c * CHUNK + ro], sv)
            return 0

        lax.fori_loop(0, NFULL, group_body, 0)
        if NREM:
            ro = NFULL * LANES
            for k in range(NREM):
                tile_v[k, pl.ds(0, LANES)] = row_acc(sbuf, dbuf, ro + k)
            sv = tile_row_sums(tile_v)
            plsc.store_scatter(out_v, [lane_ids + (c * CHUNK + ro)], sv,
                               mask=lane_ids < NREM)

    def chunk_pair(it, carry):
        c0 = it * NBUF
        for b in range(NBUF):
            c = c0 + b
            wait(c, b)
            compute_chunk(c, b)

            @pl.when(c + NBUF < NCHUNK)
            def _(c=c, b=b):
                start(c + NBUF, b)
        return carry

    lax.fori_loop(0, NCHUNK // NBUF, chunk_pair, 0)
    for c in range(NCHUNK - NCHUNK % NBUF, NCHUNK):
        b = c % NBUF
        wait(c, b)
        compute_chunk(c, b)
    pltpu.sync_copy(out_v, out_hbm.at[pl.ds(base, ROWS_PER_W)])


BN = 8192


def _tc_body(idx_ref, scale_ref, emb_ref, src_ref, dst_ref, out_ref):
    r = emb_ref[idx_ref[0], :] * scale_ref[0]
    t = src_ref[:] * dst_ref[:]
    out_ref[:] = jax.lax.dot_general(
        t, r, (((1,), (0,)), ((), ())),
        preferred_element_type=jnp.float32).reshape(BN // D, D)


def _tc_kernel(z_src, z_dst, ridx, rel_emb, scale, n_rows, row0):
    grid = ((n_rows + BN - 1) // BN,)
    out2d = pl.pallas_call(
        _tc_body,
        grid=grid,
        in_specs=[
            pl.BlockSpec(memory_space=pltpu.SMEM),
            pl.BlockSpec(memory_space=pltpu.SMEM),
            pl.BlockSpec((NUM_REL, D), lambda i: (0, 0)),
            pl.BlockSpec((BN, D), lambda i, row0=row0: (row0 // BN + i, 0)),
            pl.BlockSpec((BN, D), lambda i, row0=row0: (row0 // BN + i, 0)),
        ],
        out_specs=pl.BlockSpec((BN // D, D), lambda i: (i, 0)),
        out_shape=jax.ShapeDtypeStruct((n_rows // D, D), jnp.float32),
    )(ridx, scale, rel_emb, z_src, z_dst)
    return out2d.reshape(n_rows)


def kernel(z_src, z_dst, rel_idx, rel_emb, score_scale):
    ridx = jnp.asarray(rel_idx, jnp.int32).reshape((1,))
    scale1 = jnp.asarray(score_scale, jnp.float32).reshape((1,))
    return _tc_kernel(z_src, z_dst, ridx, rel_emb, scale1, N, 0)


def _unused_sc_kernel(z_src, z_dst, rel_idx, rel_emb, score_scale):
    ridx = jnp.asarray(rel_idx, jnp.int32).reshape((1,))
    scale = jnp.full((LANES,), score_scale, jnp.float32)
    mesh = plsc.VectorSubcoreMesh(
        core_axis_name="c", subcore_axis_name="s",
        num_cores=NC, num_subcores=NS,
    )
    f = pl.kernel(
        _body,
        out_type=jax.ShapeDtypeStruct((N,), jnp.float32),
        mesh=mesh,
        scratch_types=[
            pltpu.VMEM((1,), jnp.int32),
            pltpu.VMEM((LANES,), jnp.float32),
            pltpu.VMEM((1, D), jnp.float32),
            pltpu.VMEM((NBUF, CHUNK, D), jnp.float32),
            pltpu.VMEM((NBUF, CHUNK, D), jnp.float32),
            pltpu.VMEM((ROWS_PER_W,), jnp.float32),
            pltpu.VMEM((LANES, LANES), jnp.float32),
            pltpu.SemaphoreType.DMA((NBUF,)),
            pltpu.SemaphoreType.DMA,
        ],
        compiler_params=pltpu.CompilerParams(needs_layout_passes=False),
    )
    return f(z_src, z_dst, ridx, rel_emb, scale)
